# baseline (speedup 1.0000x reference)
import functools

import jax
import jax.numpy as jnp
from jax import lax
from jax.experimental import pallas as pl
from jax.experimental.pallas import tpu as pltpu
from jax.experimental.pallas import tpu_sc as plsc

DIM = 64
BATCH = 16384
NC = 2
NS = 16
NW = NC * NS
BPW = BATCH // NW
CHUNK = 128
NCHUNK = BPW // CHUNK
GROUPS = BPW // 16
EPS = 1e-12


def _rsqrt(x):
    i = lax.bitcast_convert_type(x, jnp.int32)
    i = jnp.int32(0x5F3759DF) - lax.shift_right_arithmetic(i, jnp.int32(1))
    y = lax.bitcast_convert_type(i, jnp.float32)
    hx = 0.5 * x
    for _ in range(3):
        y = y * (1.5 - hx * y * y)
    return y


def _body(hidx_hbm, tidx_hbm, ridx_hbm, ent_hbm, rel_hbm, out_hbm,
          hidx_v, tidx_v, ridx_v, h_rows, t_rows, r_rows, scores_v, sem):
    wid = lax.axis_index("s") * NC + lax.axis_index("c")
    crow = wid * NCHUNK

    pltpu.sync_copy(hidx_hbm.at[pl.ds(crow, NCHUNK)], hidx_v)
    pltpu.sync_copy(tidx_hbm.at[pl.ds(crow, NCHUNK)], tidx_v)
    pltpu.sync_copy(ridx_hbm.at[pl.ds(crow, NCHUNK)], ridx_v)

    copies = []
    for k in range(NCHUNK):
        rows = pl.ds(k * CHUNK, CHUNK)
        copies.append(pltpu.async_copy(ent_hbm.at[hidx_v.at[k]],
                                       h_rows.at[rows], sem))
        copies.append(pltpu.async_copy(ent_hbm.at[tidx_v.at[k]],
                                       t_rows.at[rows], sem))
        copies.append(pltpu.async_copy(rel_hbm.at[ridx_v.at[k]],
                                       r_rows.at[rows], sem))
    for c in copies:
        c.wait()

    zero = jnp.zeros((16,), jnp.float32)

    def group(g, carry):
        rows16 = g * 16 + lax.iota(jnp.int32, 16)
        shh = stt = srr = shr = sht = srt = zero
        col = jnp.zeros((16,), jnp.int32)
        one = jnp.ones((16,), jnp.int32)
        for _ in range(DIM):
            h = plsc.load_gather(h_rows, [rows16, col])
            t = plsc.load_gather(t_rows, [rows16, col])
            r = plsc.load_gather(r_rows, [rows16, col])
            shh = shh + h * h
            stt = stt + t * t
            srr = srr + r * r
            shr = shr + h * r
            sht = sht + h * t
            srt = srt + r * t
            col = col + one
        ih = 1.0 / jnp.maximum(shh * _rsqrt(shh), EPS)
        it = 1.0 / jnp.maximum(stt * _rsqrt(stt), EPS)
        ir = 1.0 / jnp.maximum(srr * _rsqrt(srr), EPS)
        s2 = (shh * ih * ih + srr * ir * ir + stt * it * it
              + 2.0 * (shr * (ih * ir) - sht * (ih * it) - srt * (ir * it)))
        s2 = jnp.maximum(s2, 0.0)
        scores_v[pl.ds(g * 16, 16)] = -(s2 * _rsqrt(s2))
        return carry

    lax.fori_loop(0, GROUPS, group, 0)
    pltpu.sync_copy(scores_v, out_hbm.at[pl.ds(wid * BPW, BPW)])


@jax.jit
def _transe_sc(hidx, tidx, ridx, ent_emb, rel_emb):
    mesh = plsc.VectorSubcoreMesh(core_axis_name="c", subcore_axis_name="s")
    f = pl.kernel(
        _body,
        out_type=jax.ShapeDtypeStruct((BATCH,), jnp.float32),
        mesh=mesh,
        compiler_params=pltpu.CompilerParams(
            needs_layout_passes=False, use_tc_tiling_on_sc=False),
        scratch_types=[
            pltpu.VMEM((NCHUNK, CHUNK), jnp.int32),
            pltpu.VMEM((NCHUNK, CHUNK), jnp.int32),
            pltpu.VMEM((NCHUNK, CHUNK), jnp.int32),
            pltpu.VMEM((BPW, DIM), jnp.float32),
            pltpu.VMEM((BPW, DIM), jnp.float32),
            pltpu.VMEM((BPW, DIM), jnp.float32),
            pltpu.VMEM((BPW,), jnp.float32),
            pltpu.SemaphoreType.DMA,
        ],
    )
    return f(hidx, tidx, ridx, ent_emb, rel_emb)


def kernel(batch, ent_emb, rel_emb):
    b = batch.astype(jnp.int32)
    hidx = b[:, 0].reshape(NW * NCHUNK, CHUNK)
    tidx = b[:, 1].reshape(NW * NCHUNK, CHUNK)
    ridx = b[:, 2].reshape(NW * NCHUNK, CHUNK)
    return _transe_sc(hidx, tidx, ridx, ent_emb, rel_emb)
